# baseline (speedup 1.0000x reference)
import functools

import numpy as np
import jax
import jax.numpy as jnp
from jax import lax
from jax.experimental import pallas as pl
from jax.experimental.pallas import tpu as pltpu
from jax.experimental.pallas import tpu_sc as plsc

_L = 2
_H = 8
_HID = 128
_DH = 16
_FF = 256
_N = 10000
_E = 320000
_ND = 128
_ED = 16

_NC = 2
_NS = 16
_NW = _NC * _NS
_CH = 128
_NCHUNK = _E // _CH
_C1 = 1000
_C2 = _NCHUNK - _C1

_EB = 6400
_NB = 1000
_NNB = _N // _NB
_N2 = 10240
_RPW = _N2 // _NS

_PREC = lax.Precision.HIGHEST

_MBLK = np.zeros((_HID, _HID), np.float32)
for _i in range(_HID):
    for _j in range(_HID):
        if _i // _DH == _j // _DH:
            _MBLK[_i, _j] = 1.0


def _ln(t, eps=1e-5):
    mu = jnp.mean(t, axis=-1, keepdims=True)
    var = jnp.mean((t - mu) * (t - mu), axis=-1, keepdims=True)
    return (t - mu) / jnp.sqrt(var + eps)


def _proj_body(x_ref, w_ref, b_ref, o_ref):
    o_ref[...] = jnp.dot(x_ref[...], w_ref[...], precision=_PREC) + b_ref[...]


def _proj(x, w, b):
    return pl.pallas_call(
        _proj_body,
        grid=(_NNB,),
        in_specs=[pl.BlockSpec((_NB, _ND), lambda i: (i, 0)),
                  pl.BlockSpec((_ND, _HID), lambda i: (0, 0)),
                  pl.BlockSpec((1, _HID), lambda i: (0, 0))],
        out_specs=pl.BlockSpec((_NB, _HID), lambda i: (i, 0)),
        out_shape=jax.ShapeDtypeStruct((_N, _HID), jnp.float32),
    )(x, w, b)


def _qkv_body(h_ref, wq_ref, wkv_ref, q_ref, kv_ref):
    h = h_ref[...]
    q_ref[...] = jnp.dot(h, wq_ref[...], precision=_PREC)
    kv_ref[...] = jnp.dot(h, wkv_ref[...], precision=_PREC)


def _qkv(h, wq, wkv):
    return pl.pallas_call(
        _qkv_body,
        grid=(_NNB,),
        in_specs=[pl.BlockSpec((_NB, _HID), lambda i: (i, 0)),
                  pl.BlockSpec((_HID, _HID), lambda i: (0, 0)),
                  pl.BlockSpec((_HID, 2 * _HID), lambda i: (0, 0))],
        out_specs=[pl.BlockSpec((_NB, _HID), lambda i: (i, 0)),
                   pl.BlockSpec((_NB, 2 * _HID), lambda i: (i, 0))],
        out_shape=[jax.ShapeDtypeStruct((_N, _HID), jnp.float32),
                   jax.ShapeDtypeStruct((_N, 2 * _HID), jnp.float32)],
    )(h, wq, wkv)


def _edge_body(qd_ref, kvs_ref, eat_ref, we_ref, mblk_ref, msg_ref, pd_ref):
    e = lax.dot_general(eat_ref[...], we_ref[...],
                        (((0,), (0,)), ((), ())), precision=_PREC)
    kvs = kvs_ref[...]
    ks = kvs[:, :_HID]
    vs = kvs[:, _HID:]
    t = qd_ref[...] * (ks + e)
    mb = mblk_ref[...]
    t_hi = t.astype(jnp.bfloat16)
    t_lo = (t - t_hi.astype(jnp.float32)).astype(jnp.bfloat16)
    s128 = (jnp.dot(t_hi, mb, preferred_element_type=jnp.float32)
            + jnp.dot(t_lo, mb, preferred_element_type=jnp.float32)) * 0.25
    p128 = jnp.exp(jnp.minimum(s128, 60.0))
    msg_ref[...] = p128 * (vs + e)
    pd_ref[...] = p128


def _edge(qd, kvs, eat, we, mblk, boff, nchunks):
    ne = nchunks * _CH
    espec = pl.BlockSpec((_EB, _HID), lambda i: (i, 0))
    return pl.pallas_call(
        _edge_body,
        grid=(ne // _EB,),
        in_specs=[espec,
                  pl.BlockSpec((_EB, 2 * _HID), lambda i: (i, 0)),
                  pl.BlockSpec((_ED, _EB), lambda i: (0, i + boff)),
                  pl.BlockSpec((_ED, _HID), lambda i: (0, 0)),
                  pl.BlockSpec((_HID, _HID), lambda i: (0, 0))],
        out_specs=[espec, espec],
        out_shape=[jax.ShapeDtypeStruct((ne, _HID), jnp.float32),
                   jax.ShapeDtypeStruct((ne, _HID), jnp.float32)],
    )(qd, kvs, eat, we, mblk.astype(jnp.bfloat16))


def _upd_body(pm1_ref, pm2_ref, pp1_ref, pp2_ref, h_ref, h0_ref, wo_ref,
              wff1_ref, bff1_ref, wff2_ref, bff2_ref, hn_ref, hf_ref):
    num = pm1_ref[...] + pm2_ref[...]
    den128 = pp1_ref[...] + pp2_ref[...]
    agg = num / (den128 + 1e-9)
    t = h_ref[...] + jnp.dot(agg, wo_ref[...], precision=_PREC) + h0_ref[...]
    t = _ln(t)
    ff = jnp.dot(jax.nn.relu(jnp.dot(t, wff1_ref[...], precision=_PREC)
                             + bff1_ref[...]),
                 wff2_ref[...], precision=_PREC) + bff2_ref[...]
    h2 = _ln(t + ff)
    hn_ref[...] = h2
    hf_ref[...] = _ln(h2)


def _update(pm1, pm2, pp1, pp2, h, h0, wo, wff1, bff1, wff2, bff2):
    nspec = pl.BlockSpec((_NB, _HID), lambda i: (i, 0))
    tspec = pl.BlockSpec((_NB, _HID), lambda i: (i, 0))
    return pl.pallas_call(
        _upd_body,
        grid=(_NNB,),
        in_specs=[tspec, tspec, tspec, tspec,
                  nspec, nspec,
                  pl.BlockSpec((_HID, _HID), lambda i: (0, 0)),
                  pl.BlockSpec((_HID, _FF), lambda i: (0, 0)),
                  pl.BlockSpec((1, _FF), lambda i: (0, 0)),
                  pl.BlockSpec((_FF, _HID), lambda i: (0, 0)),
                  pl.BlockSpec((1, _HID), lambda i: (0, 0))],
        out_specs=[nspec, nspec],
        out_shape=[jax.ShapeDtypeStruct((_N, _HID), jnp.float32)] * 2,
    )(pm1, pm2, pp1, pp2, h, h0, wo, wff1, bff1, wff2, bff2)


def _sc_mesh():
    return plsc.VectorSubcoreMesh(core_axis_name="c", subcore_axis_name="s")


def _gather2(q, kv, srcp, dstp, c0, nchunks):
    main = nchunks // _NW
    nxtra = nchunks - main * _NW
    ne = nchunks * _CH

    @functools.partial(
        pl.kernel,
        out_type=(jax.ShapeDtypeStruct((ne, _HID), jnp.float32),
                  jax.ShapeDtypeStruct((ne, 2 * _HID), jnp.float32)),
        mesh=_sc_mesh(),
        scratch_types=[pltpu.VMEM(((main + 1) * _CH,), jnp.int32),
                       pltpu.VMEM(((main + 1) * _CH,), jnp.int32),
                       pltpu.VMEM((_CH, _HID), jnp.float32),
                       pltpu.VMEM((_CH, _HID), jnp.float32),
                       pltpu.VMEM((_CH, 2 * _HID), jnp.float32),
                       pltpu.VMEM((_CH, 2 * _HID), jnp.float32),
                       pltpu.SemaphoreType.DMA,
                       pltpu.SemaphoreType.DMA,
                       pltpu.SemaphoreType.DMA,
                       pltpu.SemaphoreType.DMA,
                       pltpu.SemaphoreType.DMA,
                       pltpu.SemaphoreType.DMA,
                       pltpu.SemaphoreType.DMA,
                       pltpu.SemaphoreType.DMA],
    )
    def body(q_h, kv_h, src_h, dst_h, qd_h, kvs_h,
             ids, idd, bq0, bq1, bkv0, bkv1,
             gq0, gq1, gkv0, gkv1, wq0, wq1, wkv0, wkv1):
        c = lax.axis_index("c")
        s = lax.axis_index("s")
        w = c * _NS + s
        eb = w * (main * _CH)
        geb = c0 * _CH + eb
        pltpu.sync_copy(src_h.at[pl.ds(geb, main * _CH)],
                        ids.at[pl.ds(0, main * _CH)])
        pltpu.sync_copy(dst_h.at[pl.ds(geb, main * _CH)],
                        idd.at[pl.ds(0, main * _CH)])
        xtra = w < nxtra
        xbase = (_NW * main + w) * _CH
        gxbase = c0 * _CH + xbase

        @pl.when(xtra)
        def _():
            pltpu.sync_copy(src_h.at[pl.ds(gxbase, _CH)],
                            ids.at[pl.ds(main * _CH, _CH)])
            pltpu.sync_copy(dst_h.at[pl.ds(gxbase, _CH)],
                            idd.at[pl.ds(main * _CH, _CH)])

        def pair(j, carry):
            i0 = 2 * j * _CH
            i1 = (2 * j + 1) * _CH

            @pl.when(j > 0)
            def _():
                pltpu.make_async_copy(bq0, qd_h.at[pl.ds(eb, _CH)], wq0).wait()
                pltpu.make_async_copy(bkv0, kvs_h.at[pl.ds(eb, _CH)], wkv0).wait()
                pltpu.make_async_copy(bq1, qd_h.at[pl.ds(eb, _CH)], wq1).wait()
                pltpu.make_async_copy(bkv1, kvs_h.at[pl.ds(eb, _CH)], wkv1).wait()

            g0q = pltpu.async_copy(q_h.at[idd.at[pl.ds(i0, _CH)]], bq0, gq0)
            g0kv = pltpu.async_copy(kv_h.at[ids.at[pl.ds(i0, _CH)]], bkv0, gkv0)
            g1q = pltpu.async_copy(q_h.at[idd.at[pl.ds(i1, _CH)]], bq1, gq1)
            g1kv = pltpu.async_copy(kv_h.at[ids.at[pl.ds(i1, _CH)]], bkv1, gkv1)
            g0q.wait()
            g0kv.wait()
            pltpu.async_copy(bq0, qd_h.at[pl.ds(eb + i0, _CH)], wq0)
            pltpu.async_copy(bkv0, kvs_h.at[pl.ds(eb + i0, _CH)], wkv0)
            g1q.wait()
            g1kv.wait()
            pltpu.async_copy(bq1, qd_h.at[pl.ds(eb + i1, _CH)], wq1)
            pltpu.async_copy(bkv1, kvs_h.at[pl.ds(eb + i1, _CH)], wkv1)
            return carry

        lax.fori_loop(0, main // 2, pair, 0)
        pltpu.make_async_copy(bq0, qd_h.at[pl.ds(eb, _CH)], wq0).wait()
        pltpu.make_async_copy(bkv0, kvs_h.at[pl.ds(eb, _CH)], wkv0).wait()
        pltpu.make_async_copy(bq1, qd_h.at[pl.ds(eb, _CH)], wq1).wait()
        pltpu.make_async_copy(bkv1, kvs_h.at[pl.ds(eb, _CH)], wkv1).wait()

        if main % 2:
            it = (main - 1) * _CH
            tq = pltpu.async_copy(q_h.at[idd.at[pl.ds(it, _CH)]], bq0, gq0)
            tkv = pltpu.async_copy(kv_h.at[ids.at[pl.ds(it, _CH)]], bkv0, gkv0)
            tq.wait()
            tkv.wait()
            pltpu.sync_copy(bq0, qd_h.at[pl.ds(eb + it, _CH)])
            pltpu.sync_copy(bkv0, kvs_h.at[pl.ds(eb + it, _CH)])

        @pl.when(xtra)
        def _():
            xq = pltpu.async_copy(
                q_h.at[idd.at[pl.ds(main * _CH, _CH)]], bq0, gq0)
            xkv = pltpu.async_copy(
                kv_h.at[ids.at[pl.ds(main * _CH, _CH)]], bkv0, gkv0)
            xq.wait()
            xkv.wait()
            pltpu.sync_copy(bq0, qd_h.at[pl.ds(xbase, _CH)])
            pltpu.sync_copy(bkv0, kvs_h.at[pl.ds(xbase, _CH)])

    return body(q, kv, srcp, dstp)


def _scatter2(msg, pd, dstp, zm, c0, nchunks):
    main = nchunks // _NS
    nxtra = nchunks - main * _NS

    @functools.partial(
        pl.kernel,
        out_type=(jax.ShapeDtypeStruct((_N2, _HID), jnp.float32),
                  jax.ShapeDtypeStruct((_N2, _HID), jnp.float32)),
        mesh=_sc_mesh(),
        scratch_types=[pltpu.VMEM((_CH,), jnp.int32),
                       pltpu.VMEM((_CH,), jnp.int32),
                       pltpu.VMEM((_CH, _HID), jnp.float32),
                       pltpu.VMEM((_CH, _HID), jnp.float32),
                       pltpu.VMEM_SHARED((_N2, _HID), jnp.float32),
                       pltpu.SemaphoreType.DMA,
                       pltpu.SemaphoreType.DMA,
                       pltpu.SemaphoreType.DMA,
                       pltpu.SemaphoreType.DMA,
                       pltpu.SemaphoreType.DMA,
                       pltpu.SemaphoreType.DMA],
    )
    def body(msg_h, pd_h, dst_h, zm_h, om_h, op_h,
             i0r, i1r, bm0, bm1, shm, li0, li1, lm0, lm1, ss0, ss1):
        c = lax.axis_index("c")
        s = lax.axis_index("s")
        eb = s * (main * _CH)
        geb = c0 * _CH + eb
        r0 = s * _RPW
        pltpu.sync_copy(zm_h.at[pl.ds(r0, _RPW)], shm.at[pl.ds(r0, _RPW)])
        plsc.subcore_barrier()
        xtra = s < nxtra
        xbase = (_NS * main + s) * _CH
        gxbase = c0 * _CH + xbase

        def run(src_h, out_h):
            def pair(j, carry):
                o0 = 2 * j * _CH
                o1 = (2 * j + 1) * _CH

                @pl.when(j > 0)
                def _():
                    pltpu.make_async_copy(bm0, shm.at[i0r], ss0).wait()
                    pltpu.make_async_copy(bm1, shm.at[i1r], ss1).wait()

                a_i0 = pltpu.async_copy(
                    dst_h.at[pl.ds(geb + o0, _CH)], i0r, li0)
                a_m0 = pltpu.async_copy(
                    src_h.at[pl.ds(eb + o0, _CH)], bm0, lm0)
                a_i1 = pltpu.async_copy(
                    dst_h.at[pl.ds(geb + o1, _CH)], i1r, li1)
                a_m1 = pltpu.async_copy(
                    src_h.at[pl.ds(eb + o1, _CH)], bm1, lm1)
                a_i0.wait()
                a_m0.wait()
                pltpu.async_copy(bm0, shm.at[i0r], ss0, add=True)
                a_i1.wait()
                a_m1.wait()
                pltpu.async_copy(bm1, shm.at[i1r], ss1, add=True)
                return carry

            lax.fori_loop(0, main // 2, pair, 0)
            pltpu.make_async_copy(bm0, shm.at[i0r], ss0).wait()
            pltpu.make_async_copy(bm1, shm.at[i1r], ss1).wait()

            if main % 2:
                ot = (main - 1) * _CH
                pltpu.sync_copy(dst_h.at[pl.ds(geb + ot, _CH)], i0r)
                pltpu.sync_copy(src_h.at[pl.ds(eb + ot, _CH)], bm0)
                pltpu.sync_copy(bm0, shm.at[i0r], add=True)

            @pl.when(xtra)
            def _():
                pltpu.sync_copy(dst_h.at[pl.ds(gxbase, _CH)], i0r)
                pltpu.sync_copy(src_h.at[pl.ds(xbase, _CH)], bm0)
                pltpu.sync_copy(bm0, shm.at[i0r], add=True)

            plsc.subcore_barrier()
            pltpu.sync_copy(shm.at[pl.ds(r0, _RPW)], out_h.at[pl.ds(r0, _RPW)])

        @pl.when(c == 0)
        def _():
            run(msg_h, om_h)

        @pl.when(c == 1)
        def _():
            run(pd_h, op_h)

    return body(msg, pd, dstp, zm)


def kernel(x, edge_index, edge_attr, node_mask, edge_mask, Wproj, bproj,
           Wq, Wk, Wv, We, Wo, Wff1, bff1, Wff2, bff2):
    del node_mask, edge_mask
    srcp = edge_index[0]
    dstp = edge_index[1]
    eat = edge_attr.T
    mblk = jnp.asarray(_MBLK)
    zm = jnp.zeros((_N2, _HID), jnp.float32)

    h0 = _proj(x[0], Wproj, bproj.reshape(1, _HID))
    h = h0
    hf = h0
    for l in range(_L):
        wkv = jnp.concatenate([Wk[l], Wv[l]], axis=1)
        q, kv = _qkv(h, Wq[l], wkv)
        qd1, kvs1 = _gather2(q, kv, srcp, dstp, 0, _C1)
        qd2, kvs2 = _gather2(q, kv, srcp, dstp, _C1, _C2)
        msg1, pd1 = _edge(qd1, kvs1, eat, We[l], mblk, 0, _C1)
        msg2, pd2 = _edge(qd2, kvs2, eat, We[l], mblk,
                          _C1 * _CH // _EB, _C2)
        pm1, pp1 = _scatter2(msg1, pd1, dstp, zm, 0, _C1)
        pm2, pp2 = _scatter2(msg2, pd2, dstp, zm, _C1, _C2)
        h, hf = _update(pm1, pm2, pp1, pp2, h, h0, Wo[l],
                        Wff1[l], bff1[l].reshape(1, _FF),
                        Wff2[l], bff2[l].reshape(1, _HID))
    return hf[None]


# speedup vs baseline: 1.0174x; 1.0174x over previous
import functools

import numpy as np
import jax
import jax.numpy as jnp
from jax import lax
from jax.experimental import pallas as pl
from jax.experimental.pallas import tpu as pltpu
from jax.experimental.pallas import tpu_sc as plsc

_L = 2
_H = 8
_HID = 128
_DH = 16
_FF = 256
_N = 10000
_E = 320000
_ND = 128
_ED = 16

_NC = 2
_NS = 16
_NW = _NC * _NS
_CH = 128
_NCHUNK = _E // _CH
_C1 = 1250
_C2 = _NCHUNK - _C1

_EB = 6400
_NB = 1000
_NNB = _N // _NB
_N2 = 10240
_RPW = _N2 // _NS

_PREC = lax.Precision.HIGHEST

_MBLK = np.zeros((_HID, _HID), np.float32)
for _i in range(_HID):
    for _j in range(_HID):
        if _i // _DH == _j // _DH:
            _MBLK[_i, _j] = 1.0


def _ln(t, eps=1e-5):
    mu = jnp.mean(t, axis=-1, keepdims=True)
    var = jnp.mean((t - mu) * (t - mu), axis=-1, keepdims=True)
    return (t - mu) / jnp.sqrt(var + eps)


def _proj_body(x_ref, w_ref, b_ref, o_ref):
    o_ref[...] = jnp.dot(x_ref[...], w_ref[...], precision=_PREC) + b_ref[...]


def _proj(x, w, b):
    return pl.pallas_call(
        _proj_body,
        grid=(_NNB,),
        in_specs=[pl.BlockSpec((_NB, _ND), lambda i: (i, 0)),
                  pl.BlockSpec((_ND, _HID), lambda i: (0, 0)),
                  pl.BlockSpec((1, _HID), lambda i: (0, 0))],
        out_specs=pl.BlockSpec((_NB, _HID), lambda i: (i, 0)),
        out_shape=jax.ShapeDtypeStruct((_N, _HID), jnp.float32),
    )(x, w, b)


def _qkv_body(h_ref, wq_ref, wkv_ref, q_ref, kv_ref):
    h = h_ref[...]
    q_ref[...] = jnp.dot(h, wq_ref[...], precision=_PREC)
    kv_ref[...] = jnp.dot(h, wkv_ref[...], precision=_PREC)


def _qkv(h, wq, wkv):
    return pl.pallas_call(
        _qkv_body,
        grid=(_NNB,),
        in_specs=[pl.BlockSpec((_NB, _HID), lambda i: (i, 0)),
                  pl.BlockSpec((_HID, _HID), lambda i: (0, 0)),
                  pl.BlockSpec((_HID, 2 * _HID), lambda i: (0, 0))],
        out_specs=[pl.BlockSpec((_NB, _HID), lambda i: (i, 0)),
                   pl.BlockSpec((_NB, 2 * _HID), lambda i: (i, 0))],
        out_shape=[jax.ShapeDtypeStruct((_N, _HID), jnp.float32),
                   jax.ShapeDtypeStruct((_N, 2 * _HID), jnp.float32)],
    )(h, wq, wkv)


def _edge_body(qd_ref, kvs_ref, eat_ref, we_ref, mblk_ref, msg_ref, pd_ref):
    e = lax.dot_general(eat_ref[...], we_ref[...],
                        (((0,), (0,)), ((), ())), precision=_PREC)
    kvs = kvs_ref[...]
    ks = kvs[:, :_HID]
    vs = kvs[:, _HID:]
    t = qd_ref[...] * (ks + e)
    mb = mblk_ref[...]
    t_hi = t.astype(jnp.bfloat16)
    t_lo = (t - t_hi.astype(jnp.float32)).astype(jnp.bfloat16)
    s128 = (jnp.dot(t_hi, mb, preferred_element_type=jnp.float32)
            + jnp.dot(t_lo, mb, preferred_element_type=jnp.float32)) * 0.25
    p128 = jnp.exp(jnp.minimum(s128, 60.0))
    msg_ref[...] = p128 * (vs + e)
    pd_ref[...] = p128


def _edge(qd, kvs, eat, we, mblk, boff, nchunks):
    ne = nchunks * _CH
    espec = pl.BlockSpec((_EB, _HID), lambda i: (i, 0))
    return pl.pallas_call(
        _edge_body,
        grid=(ne // _EB,),
        in_specs=[espec,
                  pl.BlockSpec((_EB, 2 * _HID), lambda i: (i, 0)),
                  pl.BlockSpec((_ED, _EB), lambda i: (0, i + boff)),
                  pl.BlockSpec((_ED, _HID), lambda i: (0, 0)),
                  pl.BlockSpec((_HID, _HID), lambda i: (0, 0))],
        out_specs=[espec, espec],
        out_shape=[jax.ShapeDtypeStruct((ne, _HID), jnp.float32),
                   jax.ShapeDtypeStruct((ne, _HID), jnp.float32)],
    )(qd, kvs, eat, we, mblk.astype(jnp.bfloat16))


def _upd_body(pm1_ref, pm2_ref, pp1_ref, pp2_ref, h_ref, h0_ref, wo_ref,
              wff1_ref, bff1_ref, wff2_ref, bff2_ref, hn_ref, hf_ref):
    num = pm1_ref[...] + pm2_ref[...]
    den128 = pp1_ref[...] + pp2_ref[...]
    agg = num / (den128 + 1e-9)
    t = h_ref[...] + jnp.dot(agg, wo_ref[...], precision=_PREC) + h0_ref[...]
    t = _ln(t)
    ff = jnp.dot(jax.nn.relu(jnp.dot(t, wff1_ref[...], precision=_PREC)
                             + bff1_ref[...]),
                 wff2_ref[...], precision=_PREC) + bff2_ref[...]
    h2 = _ln(t + ff)
    hn_ref[...] = h2
    hf_ref[...] = _ln(h2)


def _update(pm1, pm2, pp1, pp2, h, h0, wo, wff1, bff1, wff2, bff2):
    nspec = pl.BlockSpec((_NB, _HID), lambda i: (i, 0))
    tspec = pl.BlockSpec((_NB, _HID), lambda i: (i, 0))
    return pl.pallas_call(
        _upd_body,
        grid=(_NNB,),
        in_specs=[tspec, tspec, tspec, tspec,
                  nspec, nspec,
                  pl.BlockSpec((_HID, _HID), lambda i: (0, 0)),
                  pl.BlockSpec((_HID, _FF), lambda i: (0, 0)),
                  pl.BlockSpec((1, _FF), lambda i: (0, 0)),
                  pl.BlockSpec((_FF, _HID), lambda i: (0, 0)),
                  pl.BlockSpec((1, _HID), lambda i: (0, 0))],
        out_specs=[nspec, nspec],
        out_shape=[jax.ShapeDtypeStruct((_N, _HID), jnp.float32)] * 2,
    )(pm1, pm2, pp1, pp2, h, h0, wo, wff1, bff1, wff2, bff2)


def _sc_mesh():
    return plsc.VectorSubcoreMesh(core_axis_name="c", subcore_axis_name="s")


def _gather2(q, kv, srcp, dstp, c0, nchunks):
    main = nchunks // _NW
    nxtra = nchunks - main * _NW
    ne = nchunks * _CH

    @functools.partial(
        pl.kernel,
        out_type=(jax.ShapeDtypeStruct((ne, _HID), jnp.float32),
                  jax.ShapeDtypeStruct((ne, 2 * _HID), jnp.float32)),
        mesh=_sc_mesh(),
        scratch_types=[pltpu.VMEM(((main + 1) * _CH,), jnp.int32),
                       pltpu.VMEM(((main + 1) * _CH,), jnp.int32),
                       pltpu.VMEM((_CH, _HID), jnp.float32),
                       pltpu.VMEM((_CH, _HID), jnp.float32),
                       pltpu.VMEM((_CH, 2 * _HID), jnp.float32),
                       pltpu.VMEM((_CH, 2 * _HID), jnp.float32),
                       pltpu.SemaphoreType.DMA,
                       pltpu.SemaphoreType.DMA,
                       pltpu.SemaphoreType.DMA,
                       pltpu.SemaphoreType.DMA,
                       pltpu.SemaphoreType.DMA,
                       pltpu.SemaphoreType.DMA,
                       pltpu.SemaphoreType.DMA,
                       pltpu.SemaphoreType.DMA],
    )
    def body(q_h, kv_h, src_h, dst_h, qd_h, kvs_h,
             ids, idd, bq0, bq1, bkv0, bkv1,
             gq0, gq1, gkv0, gkv1, wq0, wq1, wkv0, wkv1):
        c = lax.axis_index("c")
        s = lax.axis_index("s")
        w = c * _NS + s
        eb = w * (main * _CH)
        geb = c0 * _CH + eb
        pltpu.sync_copy(src_h.at[pl.ds(geb, main * _CH)],
                        ids.at[pl.ds(0, main * _CH)])
        pltpu.sync_copy(dst_h.at[pl.ds(geb, main * _CH)],
                        idd.at[pl.ds(0, main * _CH)])
        xtra = w < nxtra
        xbase = (_NW * main + w) * _CH
        gxbase = c0 * _CH + xbase

        @pl.when(xtra)
        def _():
            pltpu.sync_copy(src_h.at[pl.ds(gxbase, _CH)],
                            ids.at[pl.ds(main * _CH, _CH)])
            pltpu.sync_copy(dst_h.at[pl.ds(gxbase, _CH)],
                            idd.at[pl.ds(main * _CH, _CH)])

        def pair(j, carry):
            i0 = 2 * j * _CH
            i1 = (2 * j + 1) * _CH

            @pl.when(j > 0)
            def _():
                pltpu.make_async_copy(bq0, qd_h.at[pl.ds(eb, _CH)], wq0).wait()
                pltpu.make_async_copy(bkv0, kvs_h.at[pl.ds(eb, _CH)], wkv0).wait()
                pltpu.make_async_copy(bq1, qd_h.at[pl.ds(eb, _CH)], wq1).wait()
                pltpu.make_async_copy(bkv1, kvs_h.at[pl.ds(eb, _CH)], wkv1).wait()

            g0q = pltpu.async_copy(q_h.at[idd.at[pl.ds(i0, _CH)]], bq0, gq0)
            g0kv = pltpu.async_copy(kv_h.at[ids.at[pl.ds(i0, _CH)]], bkv0, gkv0)
            g1q = pltpu.async_copy(q_h.at[idd.at[pl.ds(i1, _CH)]], bq1, gq1)
            g1kv = pltpu.async_copy(kv_h.at[ids.at[pl.ds(i1, _CH)]], bkv1, gkv1)
            g0q.wait()
            g0kv.wait()
            pltpu.async_copy(bq0, qd_h.at[pl.ds(eb + i0, _CH)], wq0)
            pltpu.async_copy(bkv0, kvs_h.at[pl.ds(eb + i0, _CH)], wkv0)
            g1q.wait()
            g1kv.wait()
            pltpu.async_copy(bq1, qd_h.at[pl.ds(eb + i1, _CH)], wq1)
            pltpu.async_copy(bkv1, kvs_h.at[pl.ds(eb + i1, _CH)], wkv1)
            return carry

        lax.fori_loop(0, main // 2, pair, 0)
        pltpu.make_async_copy(bq0, qd_h.at[pl.ds(eb, _CH)], wq0).wait()
        pltpu.make_async_copy(bkv0, kvs_h.at[pl.ds(eb, _CH)], wkv0).wait()
        pltpu.make_async_copy(bq1, qd_h.at[pl.ds(eb, _CH)], wq1).wait()
        pltpu.make_async_copy(bkv1, kvs_h.at[pl.ds(eb, _CH)], wkv1).wait()

        if main % 2:
            it = (main - 1) * _CH
            tq = pltpu.async_copy(q_h.at[idd.at[pl.ds(it, _CH)]], bq0, gq0)
            tkv = pltpu.async_copy(kv_h.at[ids.at[pl.ds(it, _CH)]], bkv0, gkv0)
            tq.wait()
            tkv.wait()
            pltpu.sync_copy(bq0, qd_h.at[pl.ds(eb + it, _CH)])
            pltpu.sync_copy(bkv0, kvs_h.at[pl.ds(eb + it, _CH)])

        @pl.when(xtra)
        def _():
            xq = pltpu.async_copy(
                q_h.at[idd.at[pl.ds(main * _CH, _CH)]], bq0, gq0)
            xkv = pltpu.async_copy(
                kv_h.at[ids.at[pl.ds(main * _CH, _CH)]], bkv0, gkv0)
            xq.wait()
            xkv.wait()
            pltpu.sync_copy(bq0, qd_h.at[pl.ds(xbase, _CH)])
            pltpu.sync_copy(bkv0, kvs_h.at[pl.ds(xbase, _CH)])

    return body(q, kv, srcp, dstp)


def _scatter2(msg, pd, dstp, zm, c0, nchunks):
    main = nchunks // _NS
    nxtra = nchunks - main * _NS

    @functools.partial(
        pl.kernel,
        out_type=(jax.ShapeDtypeStruct((_N2, _HID), jnp.float32),
                  jax.ShapeDtypeStruct((_N2, _HID), jnp.float32)),
        mesh=_sc_mesh(),
        scratch_types=[pltpu.VMEM((_CH,), jnp.int32),
                       pltpu.VMEM((_CH,), jnp.int32),
                       pltpu.VMEM((_CH, _HID), jnp.float32),
                       pltpu.VMEM((_CH, _HID), jnp.float32),
                       pltpu.VMEM_SHARED((_N2, _HID), jnp.float32),
                       pltpu.SemaphoreType.DMA,
                       pltpu.SemaphoreType.DMA,
                       pltpu.SemaphoreType.DMA,
                       pltpu.SemaphoreType.DMA,
                       pltpu.SemaphoreType.DMA,
                       pltpu.SemaphoreType.DMA],
    )
    def body(msg_h, pd_h, dst_h, zm_h, om_h, op_h,
             i0r, i1r, bm0, bm1, shm, li0, li1, lm0, lm1, ss0, ss1):
        c = lax.axis_index("c")
        s = lax.axis_index("s")
        eb = s * (main * _CH)
        geb = c0 * _CH + eb
        r0 = s * _RPW
        pltpu.sync_copy(zm_h.at[pl.ds(r0, _RPW)], shm.at[pl.ds(r0, _RPW)])
        plsc.subcore_barrier()
        xtra = s < nxtra
        xbase = (_NS * main + s) * _CH
        gxbase = c0 * _CH + xbase

        def run(src_h, out_h):
            def pair(j, carry):
                o0 = 2 * j * _CH
                o1 = (2 * j + 1) * _CH

                @pl.when(j > 0)
                def _():
                    pltpu.make_async_copy(bm0, shm.at[i0r], ss0).wait()
                    pltpu.make_async_copy(bm1, shm.at[i1r], ss1).wait()

                a_i0 = pltpu.async_copy(
                    dst_h.at[pl.ds(geb + o0, _CH)], i0r, li0)
                a_m0 = pltpu.async_copy(
                    src_h.at[pl.ds(eb + o0, _CH)], bm0, lm0)
                a_i1 = pltpu.async_copy(
                    dst_h.at[pl.ds(geb + o1, _CH)], i1r, li1)
                a_m1 = pltpu.async_copy(
                    src_h.at[pl.ds(eb + o1, _CH)], bm1, lm1)
                a_i0.wait()
                a_m0.wait()
                pltpu.async_copy(bm0, shm.at[i0r], ss0, add=True)
                a_i1.wait()
                a_m1.wait()
                pltpu.async_copy(bm1, shm.at[i1r], ss1, add=True)
                return carry

            lax.fori_loop(0, main // 2, pair, 0)
            pltpu.make_async_copy(bm0, shm.at[i0r], ss0).wait()
            pltpu.make_async_copy(bm1, shm.at[i1r], ss1).wait()

            if main % 2:
                ot = (main - 1) * _CH
                pltpu.sync_copy(dst_h.at[pl.ds(geb + ot, _CH)], i0r)
                pltpu.sync_copy(src_h.at[pl.ds(eb + ot, _CH)], bm0)
                pltpu.sync_copy(bm0, shm.at[i0r], add=True)

            @pl.when(xtra)
            def _():
                pltpu.sync_copy(dst_h.at[pl.ds(gxbase, _CH)], i0r)
                pltpu.sync_copy(src_h.at[pl.ds(xbase, _CH)], bm0)
                pltpu.sync_copy(bm0, shm.at[i0r], add=True)

            plsc.subcore_barrier()
            pltpu.sync_copy(shm.at[pl.ds(r0, _RPW)], out_h.at[pl.ds(r0, _RPW)])

        @pl.when(c == 0)
        def _():
            run(msg_h, om_h)

        @pl.when(c == 1)
        def _():
            run(pd_h, op_h)

    return body(msg, pd, dstp, zm)


def kernel(x, edge_index, edge_attr, node_mask, edge_mask, Wproj, bproj,
           Wq, Wk, Wv, We, Wo, Wff1, bff1, Wff2, bff2):
    del node_mask, edge_mask
    srcp = edge_index[0]
    dstp = edge_index[1]
    eat = edge_attr.T
    mblk = jnp.asarray(_MBLK)
    zm = jnp.zeros((_N2, _HID), jnp.float32)

    h0 = _proj(x[0], Wproj, bproj.reshape(1, _HID))
    h = h0
    hf = h0
    for l in range(_L):
        wkv = jnp.concatenate([Wk[l], Wv[l]], axis=1)
        q, kv = _qkv(h, Wq[l], wkv)
        qd1, kvs1 = _gather2(q, kv, srcp, dstp, 0, _C1)
        qd2, kvs2 = _gather2(q, kv, srcp, dstp, _C1, _C2)
        msg1, pd1 = _edge(qd1, kvs1, eat, We[l], mblk, 0, _C1)
        msg2, pd2 = _edge(qd2, kvs2, eat, We[l], mblk,
                          _C1 * _CH // _EB, _C2)
        pm1, pp1 = _scatter2(msg1, pd1, dstp, zm, 0, _C1)
        pm2, pp2 = _scatter2(msg2, pd2, dstp, zm, _C1, _C2)
        h, hf = _update(pm1, pm2, pp1, pp2, h, h0, Wo[l],
                        Wff1[l], bff1[l].reshape(1, _FF),
                        Wff2[l], bff2[l].reshape(1, _HID))
    return hf[None]
